# baseline (speedup 1.0000x reference)
import jax
import jax.numpy as jnp
from jax import lax
from jax.experimental import pallas as pl
from jax.experimental.pallas import tpu as pltpu
from jax.experimental.pallas import tpu_sc as plsc

TOPK = 2048

NC = 2
NS = 16
NW = NC * NS


def _sc_pool(g1f, g2f, hf, idxf, valf, *, B, N, D, K):
    RPW = (B * K) // NW
    WPB = NW // B
    G = 8

    NG = RPW // G

    def body(g1r, g2r, hr, idxr, valr, o1r, o2r, ohr,
             colidx, rowidx, vals, inA, inB, outA, outB, hbuf,
             si0, si1, so0, so1, dsem):
        cid = lax.axis_index("c")
        sid = lax.axis_index("s")
        wid = sid * NC + cid
        b = wid // WPB
        base = wid * RPW

        ins = (inA, inB)
        outs = (outA, outB)
        sis = (si0, si1)
        sos = (so0, so1)

        pltpu.sync_copy(idxr.at[pl.ds(b * K, K)], colidx)
        pltpu.sync_copy(idxr.at[pl.ds(base, RPW)], rowidx)
        pltpu.sync_copy(valr.at[pl.ds(base, RPW)], vals)

        off = b * N

        def _addoff(i, c):
            rowidx[pl.ds(i * 16, 16)] = rowidx[pl.ds(i * 16, 16)] + off
            return c

        lax.fori_loop(0, RPW // 16, _addoff, 0)

        def in_copy(gr, grp, s):
            return pltpu.make_async_copy(
                gr.at[rowidx.at[pl.ds(grp * G, G)]], ins[s], sis[s])

        def do_g(gr, outr):
            def out_copy(grp, s):
                return pltpu.make_async_copy(
                    outs[s], outr.at[pl.ds(base + grp * G, G)], sos[s])

            in_copy(gr, 0, 0).start()
            in_copy(gr, 1, 1).start()

            def step(i, c):
                for s in (0, 1):
                    grp = i * 2 + s
                    in_copy(gr, grp, s).wait()

                    @pl.when(grp >= 2)
                    def _():
                        out_copy(grp - 2, s).wait()

                    @plsc.parallel_loop(0, K // 16, unroll=4)
                    def col(j):
                        cidx = colidx[pl.ds(j * 16, 16)]
                        for r in range(G):
                            ridx = jnp.full((16,), r, jnp.int32)
                            outs[s][r, pl.ds(j * 16, 16)] = plsc.load_gather(
                                ins[s], [ridx, cidx])
                    out_copy(grp, s).start()

                    @pl.when(grp + 2 < NG)
                    def _():
                        in_copy(gr, grp + 2, s).start()
                return c

            lax.fori_loop(0, NG // 2, step, 0)
            out_copy(NG - 2, 0).wait()
            out_copy(NG - 1, 1).wait()

        do_g(g1r, o1r)
        do_g(g2r, o2r)

        def grph(g, c):
            rowbase = g * G
            pltpu.async_copy(
                hr.at[rowidx.at[pl.ds(rowbase, G)]], hbuf, dsem).wait()
            for r in range(G):
                v16 = plsc.load_gather(
                    vals, [jnp.full((16,), rowbase + r, jnp.int32)])
                for c2 in range(D // 16):
                    sl = pl.ds(c2 * 16, 16)
                    hbuf[r, sl] = hbuf[r, sl] * v16
            pltpu.sync_copy(hbuf, ohr.at[pl.ds(base + rowbase, G)])
            return c

        lax.fori_loop(0, RPW // G, grph, 0)

    mesh = plsc.VectorSubcoreMesh(
        core_axis_name="c", subcore_axis_name="s",
        num_cores=NC, num_subcores=NS)
    f32, i32 = jnp.float32, jnp.int32
    fn = pl.kernel(
        body,
        out_type=(
            jax.ShapeDtypeStruct((B * K, K), f32),
            jax.ShapeDtypeStruct((B * K, K), f32),
            jax.ShapeDtypeStruct((B * K, D), f32),
        ),
        mesh=mesh,
        compiler_params=pltpu.CompilerParams(needs_layout_passes=False),
        scratch_types=[
            pltpu.VMEM((K,), i32),
            pltpu.VMEM((RPW,), i32),
            pltpu.VMEM((RPW,), f32),
            pltpu.VMEM((G, N), f32),
            pltpu.VMEM((G, N), f32),
            pltpu.VMEM((G, K), f32),
            pltpu.VMEM((G, K), f32),
            pltpu.VMEM((G, D), f32),
            pltpu.SemaphoreType.DMA,
            pltpu.SemaphoreType.DMA,
            pltpu.SemaphoreType.DMA,
            pltpu.SemaphoreType.DMA,
            pltpu.SemaphoreType.DMA,
        ],
    )
    return fn(g1f, g2f, hf, idxf, valf)


def _tc_row_gather(g3f, gidx, *, N, RB=8):
    BK = gidx.shape[0]

    g3v = g3f.reshape(g3f.shape[0], 1, N)

    def tc_body(gidx_ref, *refs):
        out_ref = refs[-1]
        for j in range(RB):
            out_ref[j, :] = refs[j][0, 0, :]

    def mk_imap(j):
        return lambda i, gref: (gref[i * RB + j], 0, 0)

    grid_spec = pltpu.PrefetchScalarGridSpec(
        num_scalar_prefetch=1,
        grid=(BK // RB,),
        in_specs=[pl.BlockSpec((1, 1, N), mk_imap(j)) for j in range(RB)],
        out_specs=pl.BlockSpec((RB, N), lambda i, gref: (i, 0)),
    )
    return pl.pallas_call(
        tc_body,
        grid_spec=grid_spec,
        out_shape=jax.ShapeDtypeStruct((BK, N), jnp.float32),
    )(gidx, *([g3v] * RB))


def kernel(g1, g2, g3, h, section_feature):
    B, N, D = h.shape
    k = max(2, TOPK)
    weights = jnp.matmul(h, jnp.swapaxes(section_feature, 1, 2))[:, :, 0]
    scores = jax.nn.sigmoid(weights)
    values, idx = jax.lax.top_k(scores, k)

    idx = idx.astype(jnp.int32)
    gidx = (idx + (jnp.arange(B, dtype=jnp.int32) * N)[:, None]).reshape(-1)
    o3 = _tc_row_gather(g3.reshape(B * N, N), gidx, N=N)
    o1, o2, oh = _sc_pool(
        g1.reshape(B * N, N), g2.reshape(B * N, N),
        h.reshape(B * N, D),
        idx.reshape(-1), values.reshape(-1),
        B=B, N=N, D=D, K=k)
    return (o1.reshape(B, k, k), o2.reshape(B, k, k),
            o3.reshape(B, k, N), oh.reshape(B, k, D))


# speedup vs baseline: 5.1663x; 5.1663x over previous
import jax
import jax.numpy as jnp
from jax import lax
from jax.experimental import pallas as pl
from jax.experimental.pallas import tpu as pltpu
from jax.experimental.pallas import tpu_sc as plsc

TOPK = 2048

NC = 2
NS = 16
NW = NC * NS


def _sc_pool(g1f, g2f, g3f, hf, idxf, valf, *, B, N, D, K):
    RPW = (B * K) // NW
    WPB = NW // B
    G = 8

    NG = RPW // G

    def body(g1r, g2r, g3r, hr, idxr, valr, o1r, o2r, o3r, ohr,
             colidx, rowidx, vals, inA, inB, outA, outB, hbuf,
             si0, si1, so0, so1, dsem):
        cid = lax.axis_index("c")
        sid = lax.axis_index("s")
        wid = sid * NC + cid
        b = wid // WPB
        base = wid * RPW

        ins = (inA, inB)
        outs = (outA, outB)
        sis = (si0, si1)
        sos = (so0, so1)

        pltpu.sync_copy(idxr.at[pl.ds(b * K, K)], colidx)
        pltpu.sync_copy(idxr.at[pl.ds(base, RPW)], rowidx)
        pltpu.sync_copy(valr.at[pl.ds(base, RPW)], vals)

        off = b * N

        def _addoff(i, c):
            rowidx[pl.ds(i * 16, 16)] = rowidx[pl.ds(i * 16, 16)] + off
            return c

        lax.fori_loop(0, RPW // 16, _addoff, 0)

        def in_copy(gr, grp, s):
            return pltpu.make_async_copy(
                gr.at[rowidx.at[pl.ds(grp * G, G)]], ins[s], sis[s])

        def do_g(gr, outr):
            def out_copy(grp, s):
                return pltpu.make_async_copy(
                    outs[s], outr.at[pl.ds(base + grp * G, G)], sos[s])

            in_copy(gr, 0, 0).start()
            in_copy(gr, 1, 1).start()

            def step(i, c):
                for s in (0, 1):
                    grp = i * 2 + s
                    in_copy(gr, grp, s).wait()

                    @pl.when(grp >= 2)
                    def _():
                        out_copy(grp - 2, s).wait()

                    @plsc.parallel_loop(0, K // 16, unroll=4)
                    def col(j):
                        cidx = colidx[pl.ds(j * 16, 16)]
                        for r in range(G):
                            ridx = jnp.full((16,), r, jnp.int32)
                            outs[s][r, pl.ds(j * 16, 16)] = plsc.load_gather(
                                ins[s], [ridx, cidx])
                    out_copy(grp, s).start()

                    @pl.when(grp + 2 < NG)
                    def _():
                        in_copy(gr, grp + 2, s).start()
                return c

            lax.fori_loop(0, NG // 2, step, 0)
            out_copy(NG - 2, 0).wait()
            out_copy(NG - 1, 1).wait()

        do_g(g1r, o1r)
        do_g(g2r, o2r)

        def out3_copy(grp, s):
            return pltpu.make_async_copy(
                ins[s], o3r.at[pl.ds(base + grp * G, G)], sos[s])

        in_copy(g3r, 0, 0).start()
        in_copy(g3r, 1, 1).start()

        def step3(i, c):
            for s in (0, 1):
                grp = i * 2 + s
                in_copy(g3r, grp, s).wait()
                out3_copy(grp, s).start()

                @pl.when(grp + 2 < NG)
                def _():
                    out3_copy(grp, s).wait()
                    in_copy(g3r, grp + 2, s).start()
            return c

        lax.fori_loop(0, NG // 2, step3, 0)
        out3_copy(NG - 2, 0).wait()
        out3_copy(NG - 1, 1).wait()

        def grph(g, c):
            rowbase = g * G
            pltpu.async_copy(
                hr.at[rowidx.at[pl.ds(rowbase, G)]], hbuf, dsem).wait()
            for r in range(G):
                v16 = plsc.load_gather(
                    vals, [jnp.full((16,), rowbase + r, jnp.int32)])
                for c2 in range(D // 16):
                    sl = pl.ds(c2 * 16, 16)
                    hbuf[r, sl] = hbuf[r, sl] * v16
            pltpu.sync_copy(hbuf, ohr.at[pl.ds(base + rowbase, G)])
            return c

        lax.fori_loop(0, RPW // G, grph, 0)

    mesh = plsc.VectorSubcoreMesh(
        core_axis_name="c", subcore_axis_name="s",
        num_cores=NC, num_subcores=NS)
    f32, i32 = jnp.float32, jnp.int32
    fn = pl.kernel(
        body,
        out_type=(
            jax.ShapeDtypeStruct((B * K, K), f32),
            jax.ShapeDtypeStruct((B * K, K), f32),
            jax.ShapeDtypeStruct((B * K, N), f32),
            jax.ShapeDtypeStruct((B * K, D), f32),
        ),
        mesh=mesh,
        compiler_params=pltpu.CompilerParams(needs_layout_passes=False),
        scratch_types=[
            pltpu.VMEM((K,), i32),
            pltpu.VMEM((RPW,), i32),
            pltpu.VMEM((RPW,), f32),
            pltpu.VMEM((G, N), f32),
            pltpu.VMEM((G, N), f32),
            pltpu.VMEM((G, K), f32),
            pltpu.VMEM((G, K), f32),
            pltpu.VMEM((G, D), f32),
            pltpu.SemaphoreType.DMA,
            pltpu.SemaphoreType.DMA,
            pltpu.SemaphoreType.DMA,
            pltpu.SemaphoreType.DMA,
            pltpu.SemaphoreType.DMA,
        ],
    )
    return fn(g1f, g2f, g3f, hf, idxf, valf)


def kernel(g1, g2, g3, h, section_feature):
    B, N, D = h.shape
    k = max(2, TOPK)
    weights = jnp.matmul(h, jnp.swapaxes(section_feature, 1, 2))[:, :, 0]
    scores = jax.nn.sigmoid(weights)
    values, idx = jax.lax.top_k(scores, k)

    o1, o2, o3, oh = _sc_pool(
        g1.reshape(B * N, N), g2.reshape(B * N, N), g3.reshape(B * N, N),
        h.reshape(B * N, D),
        idx.reshape(-1).astype(jnp.int32), values.reshape(-1),
        B=B, N=N, D=D, K=k)
    return (o1.reshape(B, k, k), o2.reshape(B, k, k),
            o3.reshape(B, k, N), oh.reshape(B, k, D))


# speedup vs baseline: 5.5245x; 1.0693x over previous
import jax
import jax.numpy as jnp
from jax import lax
from jax.experimental import pallas as pl
from jax.experimental.pallas import tpu as pltpu
from jax.experimental.pallas import tpu_sc as plsc

TOPK = 2048

NC = 2
NS = 16
NW = NC * NS


def _sc_pool(g1f, g2f, g3f, hf, idxf, valf, *, B, N, D, K):
    RPW = (B * K) // NW
    WPB = NW // B
    G = 8

    NG = RPW // G

    def body(g1r, g2r, g3r, hr, idxr, valr, o1r, o2r, o3r, ohr,
             colidx, rowidx, vals, inA, inB, outA, outB, hbuf,
             si0, si1, so0, so1, dsem):
        cid = lax.axis_index("c")
        sid = lax.axis_index("s")
        wid = sid * NC + cid
        b = wid // WPB
        base = wid * RPW

        ins = (inA, inB)
        outs = (outA, outB)
        sis = (si0, si1)
        sos = (so0, so1)

        pltpu.sync_copy(idxr.at[pl.ds(b * K, K)], colidx)
        pltpu.sync_copy(idxr.at[pl.ds(base, RPW)], rowidx)
        pltpu.sync_copy(valr.at[pl.ds(base, RPW)], vals)

        off = b * N

        def _addoff(i, c):
            rowidx[pl.ds(i * 16, 16)] = rowidx[pl.ds(i * 16, 16)] + off
            return c

        lax.fori_loop(0, RPW // 16, _addoff, 0)

        def in_copy(gr, grp, s):
            return pltpu.make_async_copy(
                gr.at[rowidx.at[pl.ds(grp * G, G)]], ins[s], sis[s])

        def do_g(gr, outr):
            def out_copy(grp, s):
                return pltpu.make_async_copy(
                    outs[s], outr.at[pl.ds(base + grp * G, G)], sos[s])

            in_copy(gr, 0, 0).start()
            in_copy(gr, 1, 1).start()

            def step(i, c):
                for s in (0, 1):
                    grp = i * 2 + s
                    in_copy(gr, grp, s).wait()

                    @pl.when(grp >= 2)
                    def _():
                        out_copy(grp - 2, s).wait()

                    @plsc.parallel_loop(0, K // 16, unroll=4)
                    def col(j):
                        cidx = colidx[pl.ds(j * 16, 16)]
                        for r in range(G):
                            ridx = jnp.full((16,), r, jnp.int32)
                            outs[s][r, pl.ds(j * 16, 16)] = plsc.load_gather(
                                ins[s], [ridx, cidx])
                    out_copy(grp, s).start()

                    @pl.when(grp + 2 < NG)
                    def _():
                        in_copy(gr, grp + 2, s).start()
                return c

            lax.fori_loop(0, NG // 2, step, 0)
            out_copy(NG - 2, 0).wait()
            out_copy(NG - 1, 1).wait()

        do_g(g1r, o1r)
        do_g(g2r, o2r)

        def out3_copy(grp, s):
            return pltpu.make_async_copy(
                ins[s], o3r.at[pl.ds(base + grp * G, G)], sos[s])

        in_copy(g3r, 0, 0).start()
        in_copy(g3r, 1, 1).start()

        def step3(i, c):
            for s in (0, 1):
                grp = i * 2 + s
                in_copy(g3r, grp, s).wait()
                out3_copy(grp, s).start()

                @pl.when(grp + 2 < NG)
                def _():
                    out3_copy(grp, s).wait()
                    in_copy(g3r, grp + 2, s).start()
            return c

        lax.fori_loop(0, NG // 2, step3, 0)
        out3_copy(NG - 2, 0).wait()
        out3_copy(NG - 1, 1).wait()

        HG = 64

        def grph(g, c):
            rowbase = g * HG
            pltpu.async_copy(
                hr.at[rowidx.at[pl.ds(rowbase, HG)]], hbuf, dsem).wait()

            @plsc.parallel_loop(0, HG, unroll=2)
            def _scale(r):
                v16 = plsc.load_gather(
                    vals, [jnp.full((16,), rowbase + r, jnp.int32)])
                for c2 in range(D // 16):
                    sl = pl.ds(c2 * 16, 16)
                    hbuf[r, sl] = hbuf[r, sl] * v16

            pltpu.sync_copy(hbuf, ohr.at[pl.ds(base + rowbase, HG)])
            return c

        lax.fori_loop(0, RPW // HG, grph, 0)

    mesh = plsc.VectorSubcoreMesh(
        core_axis_name="c", subcore_axis_name="s",
        num_cores=NC, num_subcores=NS)
    f32, i32 = jnp.float32, jnp.int32
    fn = pl.kernel(
        body,
        out_type=(
            jax.ShapeDtypeStruct((B * K, K), f32),
            jax.ShapeDtypeStruct((B * K, K), f32),
            jax.ShapeDtypeStruct((B * K, N), f32),
            jax.ShapeDtypeStruct((B * K, D), f32),
        ),
        mesh=mesh,
        compiler_params=pltpu.CompilerParams(needs_layout_passes=False),
        scratch_types=[
            pltpu.VMEM((K,), i32),
            pltpu.VMEM((RPW,), i32),
            pltpu.VMEM((RPW,), f32),
            pltpu.VMEM((G, N), f32),
            pltpu.VMEM((G, N), f32),
            pltpu.VMEM((G, K), f32),
            pltpu.VMEM((G, K), f32),
            pltpu.VMEM((64, D), f32),
            pltpu.SemaphoreType.DMA,
            pltpu.SemaphoreType.DMA,
            pltpu.SemaphoreType.DMA,
            pltpu.SemaphoreType.DMA,
            pltpu.SemaphoreType.DMA,
        ],
    )
    return fn(g1f, g2f, g3f, hf, idxf, valf)


def kernel(g1, g2, g3, h, section_feature):
    B, N, D = h.shape
    k = max(2, TOPK)
    weights = jnp.matmul(h, jnp.swapaxes(section_feature, 1, 2))[:, :, 0]
    scores = jax.nn.sigmoid(weights)
    values, idx = jax.lax.top_k(scores, k)

    o1, o2, o3, oh = _sc_pool(
        g1.reshape(B * N, N), g2.reshape(B * N, N), g3.reshape(B * N, N),
        h.reshape(B * N, D),
        idx.reshape(-1).astype(jnp.int32), values.reshape(-1),
        B=B, N=N, D=D, K=k)
    return (o1.reshape(B, k, k), o2.reshape(B, k, k),
            o3.reshape(B, k, N), oh.reshape(B, k, D))
